# Initial kernel scaffold; baseline (speedup 1.0000x reference)
import jax
import jax.numpy as jnp
from jax.experimental import pallas as pl


def kernel(x, W_ggl, b_ggl, emb_in, emb_out, W_gat, a_src, a_dst, b_gat, gamma, beta, W3, b3):
    raise NotImplementedError("write your pallas kernel here")


# speedup vs baseline: 112.0308x; 112.0308x over previous
import jax
import jax.numpy as jnp
from jax.experimental import pallas as pl

N = 200
H = 7
C = 300
K = 20
HI = jax.lax.Precision.HIGHEST
NEG = -1e30


def _fused(x_ref, wggl_ref, bggl_ref, ein_ref, eout_ref, wgat_ref,
           asrc_ref, adst_ref, bgat_ref, gamma_ref, beta_ref, w3_ref, b3_ref,
           out_ref):
    f32 = jnp.float32
    x = x_ref[...]

    z = jnp.dot(x, wggl_ref[...], precision=HI) + bggl_ref[...]
    atrr = 1.0 / (1.0 + jnp.exp(-z))
    A = jax.lax.dot_general(atrr, atrr, (((1,), (1,)), ((), ())), precision=HI)

    row_i = jax.lax.broadcasted_iota(jnp.int32, (N, N), 0)
    col_j = jax.lax.broadcasted_iota(jnp.int32, (N, N), 1)

    def sel_body(_, carry):
        a_work, adj = carry
        rowmax = jnp.max(a_work, axis=1, keepdims=True)
        cand = jnp.where(a_work == rowmax, col_j, N)
        jstar = jnp.min(cand, axis=1, keepdims=True)
        pick = col_j == jstar
        adj = jnp.where(pick, 1.0, adj)
        a_work = jnp.where(pick, NEG, a_work)
        return a_work, adj

    _, adj = jax.lax.fori_loop(0, K, sel_body,
                               (A, jnp.zeros((N, N), f32)))

    ones_col = jnp.ones((N, 1), f32)
    in_deg = jax.lax.dot_general(adj, ones_col, (((0,), (0,)), ((), ())),
                                 precision=HI)
    in_idx = jnp.minimum(in_deg, float(N - 1))
    onehot_in = (col_j.astype(f32) == in_idx).astype(f32)
    in_f = jnp.dot(onehot_in, ein_ref[...], precision=HI)
    onehot_out = (col_j == K).astype(f32)
    out_f = jnp.dot(onehot_out, eout_ref[...], precision=HI)

    eye_f = (row_i == col_j).astype(f32)
    dist0 = 2.0 * eye_f - 1.0

    def bfs_cond(carry):
        return carry[4] != 0

    def bfs_body(carry):
        d, visited, dist, frontier, _ = carry
        allowed = frontier * jnp.where(d < row_i, 1.0, 0.0)
        reach = jnp.dot(allowed, adj, precision=HI)
        nxt = jnp.where((reach > 0.0) & (visited == 0.0), 1.0, 0.0)
        dist = jnp.where(nxt > 0.0, (d + 1).astype(f32), dist)
        visited = visited + nxt
        go = jnp.where(jnp.any(nxt > 0.0) & (d + 1 < N),
                       jnp.int32(1), jnp.int32(0))
        return d + 1, visited, dist, nxt, go

    _, _, dist, _, _ = jax.lax.while_loop(
        bfs_cond, bfs_body,
        (jnp.int32(0), eye_f, dist0, eye_f, jnp.int32(1)))
    emask = dist != -1.0

    in_cat = jnp.concatenate([x, in_f, out_f], axis=1)
    h = jnp.dot(in_cat, wgat_ref[...], precision=HI)
    es = jnp.dot(h, asrc_ref[...], precision=HI)
    ed_t = jax.lax.dot_general(adst_ref[...], h, (((0,), (1,)), ((), ())),
                               precision=HI)

    outs = []
    for hh in range(H):
        es_col = es[:, hh:hh + 1]
        ed_row = ed_t[hh:hh + 1, :]
        v = es_col + ed_row
        logit = jnp.where(v >= 0.0, v, 0.2 * v) + dist
        logit = jnp.where(emask, logit, NEG)
        m = jnp.max(logit, axis=0, keepdims=True)
        e = jnp.exp(logit - m)
        den = jnp.sum(e, axis=0, keepdims=True)
        alpha = e / (den + 1e-16)
        hcol = h[:, hh * C:(hh + 1) * C]
        outs.append(jax.lax.dot_general(
            alpha, hcol, (((0,), (0,)), ((), ())), precision=HI))
    out = jnp.concatenate(outs, axis=1) + bgat_ref[...]

    mu = jnp.mean(out, axis=0, keepdims=True)
    ctr = out - mu
    var = jnp.mean(ctr * ctr, axis=0, keepdims=True)
    out = ctr * jax.lax.rsqrt(var + 1e-5) * gamma_ref[...] + beta_ref[...]

    res = jnp.dot(out, w3_ref[...], precision=HI) + b3_ref[...]
    out_ref[...] = jnp.maximum(res, 0.0)


def kernel(x, W_ggl, b_ggl, emb_in, emb_out, W_gat, a_src, a_dst, b_gat,
           gamma, beta, W3, b3):
    eyeH = jnp.eye(H, dtype=jnp.float32)
    as_mat = (a_src[:, :, None] * eyeH[:, None, :]).reshape(H * C, H)
    ad_mat = (a_dst[:, :, None] * eyeH[:, None, :]).reshape(H * C, H)
    return pl.pallas_call(
        _fused,
        out_shape=jax.ShapeDtypeStruct((N, 256), jnp.float32),
    )(x, W_ggl, b_ggl.reshape(1, -1), emb_in, emb_out, W_gat,
      as_mat, ad_mat, b_gat.reshape(1, -1), gamma.reshape(1, -1),
      beta.reshape(1, -1), W3, b3.reshape(1, -1))


# speedup vs baseline: 114.9191x; 1.0258x over previous
import jax
import jax.numpy as jnp
from jax.experimental import pallas as pl

N = 200
H = 7
C = 300
K = 20
HI = jax.lax.Precision.HIGHEST
NEG = -1e30


def _fused(x_ref, wggl_ref, bggl_ref, ein_ref, eout_ref, wgat_ref,
           asad_ref, bgat_ref, gamma_ref, beta_ref, w3_ref, b3_ref,
           out_ref):
    f32 = jnp.float32
    x = x_ref[...]

    z = jnp.dot(x, wggl_ref[...], precision=HI) + bggl_ref[...]
    atrr = 1.0 / (1.0 + jnp.exp(-z))
    A = jax.lax.dot_general(atrr, atrr, (((1,), (1,)), ((), ())), precision=HI)

    row_i = jax.lax.broadcasted_iota(jnp.int32, (N, N), 0)
    col_j = jax.lax.broadcasted_iota(jnp.int32, (N, N), 1)

    def sel_body(_, carry):
        a_work, adj = carry
        rowmax = jnp.max(a_work, axis=1, keepdims=True)
        cand = jnp.where(a_work == rowmax, col_j, N)
        jstar = jnp.min(cand, axis=1, keepdims=True)
        pick = col_j == jstar
        adj = jnp.where(pick, 1.0, adj)
        a_work = jnp.where(pick, NEG, a_work)
        return a_work, adj

    _, adj = jax.lax.fori_loop(0, K, sel_body,
                               (A, jnp.zeros((N, N), f32)))

    ones_col = jnp.ones((N, 1), f32)
    in_deg = jax.lax.dot_general(adj, ones_col, (((0,), (0,)), ((), ())),
                                 precision=HI)
    in_idx = jnp.minimum(in_deg, float(N - 1))
    onehot_in = (col_j.astype(f32) == in_idx).astype(f32)
    in_f = jnp.dot(onehot_in, ein_ref[...], precision=HI)
    onehot_out = (col_j == K).astype(f32)
    out_f = jnp.dot(onehot_out, eout_ref[...], precision=HI)

    eye_f = (row_i == col_j).astype(f32)
    dist0 = 2.0 * eye_f - 1.0

    def bfs_cond(carry):
        return carry[4] != 0

    def bfs_body(carry):
        d, visited, dist, frontier, _ = carry
        allowed = frontier * jnp.where(d < row_i, 1.0, 0.0)
        reach = jnp.dot(allowed, adj)
        nxt = jnp.where((reach > 0.0) & (visited == 0.0), 1.0, 0.0)
        dist = jnp.where(nxt > 0.0, (d + 1).astype(f32), dist)
        visited = visited + nxt
        go = jnp.where(jnp.any(nxt > 0.0) & (d + 1 < N),
                       jnp.int32(1), jnp.int32(0))
        return d + 1, visited, dist, nxt, go

    _, _, dist, _, _ = jax.lax.while_loop(
        bfs_cond, bfs_body,
        (jnp.int32(0), eye_f, dist0, eye_f, jnp.int32(1)))
    emask = dist != -1.0

    in_cat = jnp.concatenate([x, in_f, out_f], axis=1)
    h = jnp.dot(in_cat, wgat_ref[...], precision=HI)
    t2 = jax.lax.dot_general(asad_ref[...], h, (((0,), (1,)), ((), ())),
                             precision=HI)
    es = jnp.transpose(t2[:H, :])
    ed_t = t2[H:, :]

    outs = []
    for hh in range(H):
        es_col = es[:, hh:hh + 1]
        ed_row = ed_t[hh:hh + 1, :]
        v = es_col + ed_row
        logit = jnp.where(v >= 0.0, v, 0.2 * v) + dist
        logit = jnp.where(emask, logit, NEG)
        m = jnp.max(logit, axis=0, keepdims=True)
        e = jnp.exp(logit - m)
        den = jnp.sum(e, axis=0, keepdims=True)
        alpha = e / (den + 1e-16)
        hcol = h[:, hh * C:(hh + 1) * C]
        outs.append(jax.lax.dot_general(
            alpha, hcol, (((0,), (0,)), ((), ())), precision=HI))
    out = jnp.concatenate(outs, axis=1) + bgat_ref[...]

    mu = jnp.mean(out, axis=0, keepdims=True)
    ctr = out - mu
    var = jnp.mean(ctr * ctr, axis=0, keepdims=True)
    out = ctr * jax.lax.rsqrt(var + 1e-5) * gamma_ref[...] + beta_ref[...]

    res = jnp.dot(out, w3_ref[...], precision=HI) + b3_ref[...]
    out_ref[...] = jnp.maximum(res, 0.0)


def kernel(x, W_ggl, b_ggl, emb_in, emb_out, W_gat, a_src, a_dst, b_gat,
           gamma, beta, W3, b3):
    eyeH = jnp.eye(H, dtype=jnp.float32)
    as_mat = (a_src[:, :, None] * eyeH[:, None, :]).reshape(H * C, H)
    ad_mat = (a_dst[:, :, None] * eyeH[:, None, :]).reshape(H * C, H)
    asad_mat = jnp.concatenate([as_mat, ad_mat], axis=1)
    return pl.pallas_call(
        _fused,
        out_shape=jax.ShapeDtypeStruct((N, 256), jnp.float32),
    )(x, W_ggl, b_ggl.reshape(1, -1), emb_in, emb_out, W_gat,
      asad_mat, b_gat.reshape(1, -1), gamma.reshape(1, -1),
      beta.reshape(1, -1), W3, b3.reshape(1, -1))


# speedup vs baseline: 125.5223x; 1.0923x over previous
import jax
import jax.numpy as jnp
from jax.experimental import pallas as pl

N = 200
H = 7
C = 300
K = 20
HI = jax.lax.Precision.HIGHEST
NEG = -1e30


def _fused(x_ref, wggl_ref, bggl_ref, ein_ref, eout_ref, wgat_ref,
           asad_ref, bgat_ref, gamma_ref, beta_ref, w3_ref, b3_ref,
           out_ref):
    f32 = jnp.float32
    x = x_ref[...]

    z = jnp.dot(x, wggl_ref[...], precision=HI) + bggl_ref[...]
    atrr = 1.0 / (1.0 + jnp.exp(-z))
    A = jax.lax.dot_general(atrr, atrr, (((1,), (1,)), ((), ())), precision=HI)

    row_i = jax.lax.broadcasted_iota(jnp.int32, (N, N), 0)
    col_j = jax.lax.broadcasted_iota(jnp.int32, (N, N), 1)

    def sel_body(_, carry):
        a_work, adjt = carry
        colmax = jnp.max(a_work, axis=0, keepdims=True)
        cand = jnp.where(a_work == colmax, row_i, N)
        jstar = jnp.min(cand, axis=0, keepdims=True)
        pick = row_i == jstar
        adjt = jnp.where(pick, 1.0, adjt)
        a_work = jnp.where(pick, NEG, a_work)
        return a_work, adjt

    _, adjt = jax.lax.fori_loop(0, K, sel_body,
                                (A, jnp.zeros((N, N), f32)))

    ones_col = jnp.ones((N, 1), f32)
    in_deg = jnp.dot(adjt, ones_col, precision=HI)
    in_idx = jnp.minimum(in_deg, float(N - 1))
    onehot_in = (col_j.astype(f32) == in_idx).astype(f32)
    in_f = jnp.dot(onehot_in, ein_ref[...], precision=HI)
    onehot_out = (col_j == K).astype(f32)
    out_f = jnp.dot(onehot_out, eout_ref[...], precision=HI)

    eye_f = (row_i == col_j).astype(f32)
    dist0 = 2.0 * eye_f - 1.0

    def bfs_cond(carry):
        return carry[4] != 0

    def bfs_body(carry):
        d, visited, dist, frontier, _ = carry
        allowed = frontier * jnp.where(d < row_i, 1.0, 0.0)
        reach = jax.lax.dot_general(allowed, adjt, (((1,), (1,)), ((), ())))
        nxt = jnp.where((reach > 0.0) & (visited == 0.0), 1.0, 0.0)
        dist = jnp.where(nxt > 0.0, (d + 1).astype(f32), dist)
        visited = visited + nxt
        go = jnp.where(jnp.any(nxt > 0.0) & (d + 1 < N),
                       jnp.int32(1), jnp.int32(0))
        return d + 1, visited, dist, nxt, go

    _, _, dist, _, _ = jax.lax.while_loop(
        bfs_cond, bfs_body,
        (jnp.int32(0), eye_f, dist0, eye_f, jnp.int32(1)))
    emask = dist != -1.0

    in_cat = jnp.concatenate([x, in_f, out_f], axis=1)
    h = jnp.dot(in_cat, wgat_ref[...], precision=HI)
    t2 = jax.lax.dot_general(asad_ref[...], h, (((0,), (1,)), ((), ())),
                             precision=HI)
    es = jnp.transpose(t2[:H, :])
    ed_t = t2[H:, :]

    outs = []
    for hh in range(H):
        es_col = es[:, hh:hh + 1]
        ed_row = ed_t[hh:hh + 1, :]
        v = es_col + ed_row
        logit = jnp.where(v >= 0.0, v, 0.2 * v) + dist
        logit = jnp.where(emask, logit, NEG)
        m = jnp.max(logit, axis=0, keepdims=True)
        e = jnp.exp(logit - m)
        den = jnp.sum(e, axis=0, keepdims=True)
        alpha = e / (den + 1e-16)
        hcol = h[:, hh * C:(hh + 1) * C]
        outs.append(jax.lax.dot_general(
            alpha, hcol, (((0,), (0,)), ((), ())), precision=HI))
    out = jnp.concatenate(outs, axis=1) + bgat_ref[...]

    mu = jnp.mean(out, axis=0, keepdims=True)
    ctr = out - mu
    var = jnp.mean(ctr * ctr, axis=0, keepdims=True)
    out = ctr * jax.lax.rsqrt(var + 1e-5) * gamma_ref[...] + beta_ref[...]

    res = jnp.dot(out, w3_ref[...], precision=HI) + b3_ref[...]
    out_ref[...] = jnp.maximum(res, 0.0)


def kernel(x, W_ggl, b_ggl, emb_in, emb_out, W_gat, a_src, a_dst, b_gat,
           gamma, beta, W3, b3):
    eyeH = jnp.eye(H, dtype=jnp.float32)
    as_mat = (a_src[:, :, None] * eyeH[:, None, :]).reshape(H * C, H)
    ad_mat = (a_dst[:, :, None] * eyeH[:, None, :]).reshape(H * C, H)
    asad_mat = jnp.concatenate([as_mat, ad_mat], axis=1)
    return pl.pallas_call(
        _fused,
        out_shape=jax.ShapeDtypeStruct((N, 256), jnp.float32),
    )(x, W_ggl, b_ggl.reshape(1, -1), emb_in, emb_out, W_gat,
      asad_mat, b_gat.reshape(1, -1), gamma.reshape(1, -1),
      beta.reshape(1, -1), W3, b3.reshape(1, -1))


# speedup vs baseline: 134.6785x; 1.0729x over previous
import jax
import jax.numpy as jnp
from jax.experimental import pallas as pl

N = 200
H = 7
C = 300
K = 20
HI = jax.lax.Precision.HIGHEST
NEG = -1e30


def _fused(x_ref, wggl_ref, bggl_ref, ein_ref, eout_ref, wgat_ref,
           asad_ref, bgat_ref, gamma_ref, beta_ref, w3_ref, b3_ref,
           out_ref):
    f32 = jnp.float32
    x = x_ref[...]

    z = jnp.dot(x, wggl_ref[...], precision=HI) + bggl_ref[...]
    atrr = 1.0 / (1.0 + jnp.exp(-z))
    A = jax.lax.dot_general(atrr, atrr, (((1,), (1,)), ((), ())), precision=HI)

    row_i = jax.lax.broadcasted_iota(jnp.int32, (N, N), 0)
    col_j = jax.lax.broadcasted_iota(jnp.int32, (N, N), 1)

    h_x = jnp.dot(x, wgat_ref[:256, :], precision=HI)

    a_work = A
    adjt = jnp.zeros((N, N), f32)
    for _ in range(K):
        colmax = jnp.max(a_work, axis=0, keepdims=True)
        cand = jnp.where(a_work == colmax, row_i, N)
        jstar = jnp.min(cand, axis=0, keepdims=True)
        pick = row_i == jstar
        adjt = jnp.where(pick, 1.0, adjt)
        a_work = jnp.where(pick, NEG, a_work)

    ones_col = jnp.ones((N, 1), f32)
    in_deg = jnp.dot(adjt, ones_col, precision=HI)
    in_idx = jnp.minimum(in_deg, float(N - 1))
    onehot_in = (col_j.astype(f32) == in_idx).astype(f32)
    in_f = jnp.dot(onehot_in, ein_ref[...], precision=HI)
    onehot_out = (col_j == K).astype(f32)
    out_f = jnp.dot(onehot_out, eout_ref[...], precision=HI)

    h = (h_x + jnp.dot(in_f, wgat_ref[256:264, :], precision=HI)
         + jnp.dot(out_f, wgat_ref[264:272, :], precision=HI))
    t2 = jax.lax.dot_general(asad_ref[...], h, (((0,), (1,)), ((), ())),
                             precision=HI)
    es = jnp.transpose(t2[:H, :])
    ed_t = t2[H:, :]

    eye_f = (row_i == col_j).astype(f32)
    dist0 = 2.0 * eye_f - 1.0

    def bfs_cond(carry):
        return carry[3] != 0

    def bfs_body(carry):
        d, dist, frontier, _ = carry
        allowed = frontier * jnp.where(d < row_i, 1.0, 0.0)
        reach = jax.lax.dot_general(allowed, adjt, (((1,), (1,)), ((), ())))
        nxt = jnp.where((reach > 0.0) & (dist == -1.0), 1.0, 0.0)
        dist = jnp.where(nxt > 0.0, (d + 1).astype(f32), dist)
        go = jnp.where(jnp.any(nxt > 0.0) & (d + 1 < N),
                       jnp.int32(1), jnp.int32(0))
        return d + 1, dist, nxt, go

    _, dist, _, _ = jax.lax.while_loop(
        bfs_cond, bfs_body,
        (jnp.int32(0), dist0, eye_f, jnp.int32(1)))
    bias = jnp.where(dist != -1.0, dist, NEG)

    outs = []
    for hh in range(H):
        es_col = es[:, hh:hh + 1]
        ed_row = ed_t[hh:hh + 1, :]
        v = es_col + ed_row
        logit = jnp.where(v >= 0.0, v, 0.2 * v) + bias
        m = jnp.max(logit, axis=0, keepdims=True)
        e = jnp.exp(logit - m)
        den = jnp.sum(e, axis=0, keepdims=True)
        alpha = e / (den + 1e-16)
        hcol = h[:, hh * C:(hh + 1) * C]
        outs.append(jax.lax.dot_general(
            alpha, hcol, (((0,), (0,)), ((), ())), precision=HI))
    out = jnp.concatenate(outs, axis=1) + bgat_ref[...]

    mu = jnp.mean(out, axis=0, keepdims=True)
    ctr = out - mu
    var = jnp.mean(ctr * ctr, axis=0, keepdims=True)
    out = ctr * jax.lax.rsqrt(var + 1e-5) * gamma_ref[...] + beta_ref[...]

    res = jnp.dot(out, w3_ref[...], precision=HI) + b3_ref[...]
    out_ref[...] = jnp.maximum(res, 0.0)


def kernel(x, W_ggl, b_ggl, emb_in, emb_out, W_gat, a_src, a_dst, b_gat,
           gamma, beta, W3, b3):
    eyeH = jnp.eye(H, dtype=jnp.float32)
    as_mat = (a_src[:, :, None] * eyeH[:, None, :]).reshape(H * C, H)
    ad_mat = (a_dst[:, :, None] * eyeH[:, None, :]).reshape(H * C, H)
    asad_mat = jnp.concatenate([as_mat, ad_mat], axis=1)
    return pl.pallas_call(
        _fused,
        out_shape=jax.ShapeDtypeStruct((N, 256), jnp.float32),
    )(x, W_ggl, b_ggl.reshape(1, -1), emb_in, emb_out, W_gat,
      asad_mat, b_gat.reshape(1, -1), gamma.reshape(1, -1),
      beta.reshape(1, -1), W3, b3.reshape(1, -1))


# speedup vs baseline: 167.9678x; 1.2472x over previous
import jax
import jax.numpy as jnp
from jax.experimental import pallas as pl

N = 200
H = 7
C = 300
K = 20
HI = jax.lax.Precision.HIGHEST
NEG = -1e30


def _fused(x_ref, wggl_ref, bggl_ref, ein_ref, eout_ref, wgat_ref,
           asad_ref, bgat_ref, gamma_ref, beta_ref, w3_ref, b3_ref,
           out_ref):
    f32 = jnp.float32
    x = x_ref[...]

    z = jnp.dot(x, wggl_ref[...], precision=HI) + bggl_ref[...]
    atrr = 1.0 / (1.0 + jnp.exp(-z))
    A = jax.lax.dot_general(atrr, atrr, (((1,), (1,)), ((), ())), precision=HI)

    row_i = jax.lax.broadcasted_iota(jnp.int32, (N, N), 0)
    col_j = jax.lax.broadcasted_iota(jnp.int32, (N, N), 1)

    h_x = jnp.dot(x, wgat_ref[:256, :])

    a_work = A
    adjt = jnp.zeros((N, N), f32)
    for _ in range(K):
        colmax = jnp.max(a_work, axis=0, keepdims=True)
        cand = jnp.where(a_work == colmax, row_i, N)
        jstar = jnp.min(cand, axis=0, keepdims=True)
        pick = row_i == jstar
        adjt = jnp.where(pick, 1.0, adjt)
        a_work = jnp.where(pick, NEG, a_work)

    ones_col = jnp.ones((N, 1), f32)
    in_deg = jnp.dot(adjt, ones_col, precision=HI)
    in_idx = jnp.minimum(in_deg, float(N - 1))
    onehot_in = (col_j.astype(f32) == in_idx).astype(f32)
    in_f = jnp.dot(onehot_in, ein_ref[...], precision=HI)
    onehot_out = (col_j == K).astype(f32)
    out_f = jnp.dot(onehot_out, eout_ref[...], precision=HI)

    h = (h_x + jnp.dot(in_f, wgat_ref[256:264, :], precision=HI)
         + jnp.dot(out_f, wgat_ref[264:272, :], precision=HI))
    t2 = jax.lax.dot_general(asad_ref[...], h, (((0,), (1,)), ((), ())))
    es = jnp.transpose(t2[:H, :])
    ed_t = t2[H:, :]

    eye_f = (row_i == col_j).astype(f32)
    dist0 = 2.0 * eye_f - 1.0

    def bfs_cond(carry):
        return carry[3] != 0

    def bfs_body(carry):
        d, dist, frontier, _ = carry
        allowed = frontier * jnp.where(d < row_i, 1.0, 0.0)
        reach = jax.lax.dot_general(allowed, adjt, (((1,), (1,)), ((), ())))
        nxt = jnp.where((reach > 0.0) & (dist == -1.0), 1.0, 0.0)
        dist = jnp.where(nxt > 0.0, (d + 1).astype(f32), dist)
        go = jnp.where(jnp.any(nxt > 0.0) & (d + 1 < N),
                       jnp.int32(1), jnp.int32(0))
        return d + 1, dist, nxt, go

    _, dist, _, _ = jax.lax.while_loop(
        bfs_cond, bfs_body,
        (jnp.int32(0), dist0, eye_f, jnp.int32(1)))
    bias = jnp.where(dist != -1.0, dist, NEG)

    outs = []
    for hh in range(H):
        es_col = es[:, hh:hh + 1]
        ed_row = ed_t[hh:hh + 1, :]
        v = es_col + ed_row
        logit = jnp.where(v >= 0.0, v, 0.2 * v) + bias
        m = jnp.max(logit, axis=0, keepdims=True)
        e = jnp.exp(logit - m)
        den = jnp.sum(e, axis=0, keepdims=True)
        alpha = e / (den + 1e-16)
        hcol = h[:, hh * C:(hh + 1) * C]
        outs.append(jax.lax.dot_general(
            alpha, hcol, (((0,), (0,)), ((), ()))))
    out = jnp.concatenate(outs, axis=1) + bgat_ref[...]

    mu = jnp.mean(out, axis=0, keepdims=True)
    ctr = out - mu
    var = jnp.mean(ctr * ctr, axis=0, keepdims=True)
    out = ctr * jax.lax.rsqrt(var + 1e-5) * gamma_ref[...] + beta_ref[...]

    res = jnp.dot(out, w3_ref[...]) + b3_ref[...]
    out_ref[...] = jnp.maximum(res, 0.0)


def kernel(x, W_ggl, b_ggl, emb_in, emb_out, W_gat, a_src, a_dst, b_gat,
           gamma, beta, W3, b3):
    eyeH = jnp.eye(H, dtype=jnp.float32)
    as_mat = (a_src[:, :, None] * eyeH[:, None, :]).reshape(H * C, H)
    ad_mat = (a_dst[:, :, None] * eyeH[:, None, :]).reshape(H * C, H)
    asad_mat = jnp.concatenate([as_mat, ad_mat], axis=1)
    return pl.pallas_call(
        _fused,
        out_shape=jax.ShapeDtypeStruct((N, 256), jnp.float32),
    )(x, W_ggl, b_ggl.reshape(1, -1), emb_in, emb_out, W_gat,
      asad_mat, b_gat.reshape(1, -1), gamma.reshape(1, -1),
      beta.reshape(1, -1), W3, b3.reshape(1, -1))


# speedup vs baseline: 195.6846x; 1.1650x over previous
import jax
import jax.numpy as jnp
from jax.experimental import pallas as pl

N = 200
H = 7
C = 300
K = 20
HI = jax.lax.Precision.HIGHEST
NEG = -1e30


def _fused(x_ref, wggl_ref, bggl_ref, ein_ref, eout_ref, wgat_ref,
           asad_ref, bgat_ref, gamma_ref, beta_ref, w3_ref, b3_ref,
           out_ref):
    f32 = jnp.float32
    x = x_ref[...]

    z = jnp.dot(x, wggl_ref[...], precision=HI) + bggl_ref[...]
    atrr = 1.0 / (1.0 + jnp.exp(-z))
    A = jax.lax.dot_general(atrr, atrr, (((1,), (1,)), ((), ())), precision=HI)

    row_i = jax.lax.broadcasted_iota(jnp.int32, (N, N), 0)
    col_j = jax.lax.broadcasted_iota(jnp.int32, (N, N), 1)

    a_work = A
    for _ in range(K):
        colmax = jnp.max(a_work, axis=0, keepdims=True)
        cand = jnp.where(a_work == colmax, row_i, N)
        jstar = jnp.min(cand, axis=0, keepdims=True)
        a_work = jnp.where(row_i == jstar, NEG, a_work)
    adjt = jnp.where(a_work == NEG, 1.0, 0.0)

    ones_col = jnp.ones((N, 1), f32)
    in_deg = jnp.dot(adjt, ones_col, precision=HI)
    in_idx = jnp.minimum(in_deg, float(N - 1))
    onehot_in = (col_j.astype(f32) == in_idx).astype(f32)
    in_f = jnp.dot(onehot_in, ein_ref[...], precision=HI)
    onehot_out = (col_j[:1, :] == K).astype(f32)
    orow = jnp.dot(onehot_out, eout_ref[...], precision=HI)
    out_f = jnp.broadcast_to(orow, (N, 8))

    in_cat = jnp.concatenate([x, in_f, out_f], axis=1)
    h = jnp.dot(in_cat, wgat_ref[...])
    t2 = jax.lax.dot_general(asad_ref[...], h, (((0,), (1,)), ((), ())))
    es = jnp.transpose(t2[:H, :])
    ed_t = t2[H:, :]

    eye_f = (row_i == col_j).astype(f32)
    dist0 = 2.0 * eye_f - 1.0

    iota_col = jax.lax.broadcasted_iota(jnp.int32, (N, 1), 0)

    def bfs_cond(carry):
        return carry[3] != 0

    def _hop(d, dist, frontier):
        allowed = frontier * jnp.where(d < iota_col, 1.0, 0.0)
        reach = jax.lax.dot_general(allowed, adjt, (((1,), (1,)), ((), ())))
        nxt = jnp.where((reach > 0.0) & (dist == -1.0), 1.0, 0.0)
        dist = jnp.where(nxt > 0.0, (d + 1).astype(f32), dist)
        return dist, nxt

    def bfs_body(carry):
        d, dist, frontier, _ = carry
        dist, nxt = _hop(d, dist, frontier)
        dist, nxt = _hop(d + 1, dist, nxt)
        go = jnp.where(jnp.any(nxt > 0.0), jnp.int32(1), jnp.int32(0))
        return d + 2, dist, nxt, go

    _, dist, _, _ = jax.lax.while_loop(
        bfs_cond, bfs_body,
        (jnp.int32(0), dist0, eye_f, jnp.int32(1)))
    bias = jnp.where(dist != -1.0, dist, NEG)

    outs = []
    for hh in range(H):
        es_col = es[:, hh:hh + 1]
        ed_row = ed_t[hh:hh + 1, :]
        v = es_col + ed_row
        logit = jnp.maximum(v, 0.2 * v) + bias
        m = jnp.max(logit, axis=0, keepdims=True)
        e = jnp.exp(logit - m)
        den = jnp.sum(e, axis=0, keepdims=True)
        alpha = e * (1.0 / (den + 1e-16))
        hcol = h[:, hh * C:(hh + 1) * C]
        outs.append(jax.lax.dot_general(
            alpha, hcol, (((0,), (0,)), ((), ()))))
    out = jnp.concatenate(outs, axis=1) + bgat_ref[...]

    mu = jnp.mean(out, axis=0, keepdims=True)
    ctr = out - mu
    var = jnp.mean(ctr * ctr, axis=0, keepdims=True)
    out = ctr * jax.lax.rsqrt(var + 1e-5) * gamma_ref[...] + beta_ref[...]

    res = jnp.dot(out, w3_ref[...]) + b3_ref[...]
    out_ref[...] = jnp.maximum(res, 0.0)


def kernel(x, W_ggl, b_ggl, emb_in, emb_out, W_gat, a_src, a_dst, b_gat,
           gamma, beta, W3, b3):
    eyeH = jnp.eye(H, dtype=jnp.float32)
    as_mat = (a_src[:, :, None] * eyeH[:, None, :]).reshape(H * C, H)
    ad_mat = (a_dst[:, :, None] * eyeH[:, None, :]).reshape(H * C, H)
    asad_mat = jnp.concatenate([as_mat, ad_mat], axis=1)
    return pl.pallas_call(
        _fused,
        out_shape=jax.ShapeDtypeStruct((N, 256), jnp.float32),
    )(x, W_ggl, b_ggl.reshape(1, -1), emb_in, emb_out, W_gat,
      asad_mat, b_gat.reshape(1, -1), gamma.reshape(1, -1),
      beta.reshape(1, -1), W3, b3.reshape(1, -1))
